# baseline (speedup 1.0000x reference)
import jax
import jax.numpy as jnp
from jax import lax
from jax.experimental import pallas as pl
from jax.experimental.pallas import tpu as pltpu
from jax.experimental.pallas import tpu_sc as plsc

B = 1024
S = 256
H = 14
W = 14
C = 128
Q = H * W
QT = 136
NR = Q * B
NC = 2
NS = 16
NW = NC * NS
SCR = (Q - QT) * B
RPW = SCR // NW
RCH = 64
NSTEP = RPW // RCH
LANES = 16
CV = C // LANES


def _sc_kernel(inj2, msk2, x2, indices):
    mesh = plsc.VectorSubcoreMesh(core_axis_name="c", subcore_axis_name="s")

    def run(inj_hbm, msk_hbm, x_hbm, idx_hbm, out_hbm,
            idx_v, exp_v, x_v, m_v, a_v, o_v, sems, sem_wb):
        wid = lax.axis_index("s") * NC + lax.axis_index("c")
        base = QT * B + wid * RPW
        lbase = wid * RPW
        pltpu.sync_copy(idx_hbm, idx_v)

        lane = lax.iota(jnp.int32, LANES)

        @pl.loop(0, RPW, step=LANES)
        def _(t):
            r16 = base + t + lane
            q = lax.shift_right_logical(r16, 10)
            b = lax.bitwise_and(r16, 1023)
            v = plsc.load_gather(idx_v, [b])
            exp_v[pl.ds(t, LANES)] = lax.shift_left(q, 8) + v

        def issue(n, j):
            idxs = exp_v.at[pl.ds(n * RCH, RCH)]
            pltpu.async_copy(inj_hbm.at[idxs], a_v.at[j], sems.at[j, 0])
            pltpu.async_copy(msk_hbm.at[idxs], m_v.at[j], sems.at[j, 1])
            pltpu.async_copy(x_hbm.at[pl.ds(base + n * RCH, RCH)], x_v.at[j],
                             sems.at[j, 2])

        def wait_in(j):
            pltpu.make_async_copy(inj_hbm.at[pl.ds(0, RCH)], a_v.at[j],
                                  sems.at[j, 0]).wait()
            pltpu.make_async_copy(msk_hbm.at[pl.ds(0, RCH)], m_v.at[j],
                                  sems.at[j, 1]).wait()
            pltpu.make_async_copy(x_hbm.at[pl.ds(0, RCH)], x_v.at[j],
                                  sems.at[j, 2]).wait()

        def compute(j):
            xp, mp, ap = x_v.at[j], m_v.at[j], a_v.at[j]

            @pl.loop(0, RCH)
            def _(i):
                for c in range(CV):
                    sl = (i, pl.ds(c * LANES, LANES))
                    o_v.at[*sl][...] = (xp.at[*sl][...] * mp.at[*sl][...]
                                        + ap.at[*sl][...])

        def wb_start(n):
            pltpu.async_copy(o_v, out_hbm.at[pl.ds(lbase + n * RCH, RCH)],
                             sem_wb)

        def wb_wait():
            pltpu.make_async_copy(o_v, out_hbm.at[pl.ds(0, RCH)],
                                  sem_wb).wait()

        def step(n, j, wait_prev_wb, issue_next):
            wait_in(j)
            if wait_prev_wb:
                wb_wait()
            compute(j)
            wb_start(n)
            if issue_next:
                issue(n + 2, j)

        issue(0, 0)
        issue(1, 1)
        step(0, 0, False, True)
        step(1, 1, True, True)

        @pl.loop(2, NSTEP - 2, step=2)
        def _(n0):
            step(n0, 0, True, True)
            step(n0 + 1, 1, True, True)

        step(NSTEP - 2, 0, True, False)
        step(NSTEP - 1, 1, True, False)
        wb_wait()

    grid_kernel = pl.kernel(
        run,
        out_type=jax.ShapeDtypeStruct((SCR, C), jnp.float32),
        mesh=mesh,
        scratch_types=[
            pltpu.VMEM((B,), jnp.int32),
            pltpu.VMEM((RPW,), jnp.int32),
            pltpu.VMEM((2, RCH, C), jnp.float32),
            pltpu.VMEM((2, RCH, C), jnp.float32),
            pltpu.VMEM((2, RCH, C), jnp.float32),
            pltpu.VMEM((RCH, C), jnp.float32),
            pltpu.SemaphoreType.DMA((2, 3)),
            pltpu.SemaphoreType.DMA,
        ],
        compiler_params=pltpu.CompilerParams(use_tc_tiling_on_sc=True,
                                             needs_layout_passes=False),
    )
    return grid_kernel(inj2, msk2, x2, indices)


def _tc_body(oh_ref, x_ref, inj_ref, msk_ref, out_ref):
    oh = oh_ref[...]
    m = jnp.dot(oh, msk_ref[0], preferred_element_type=jnp.float32,
                precision=lax.Precision.HIGHEST)
    a = jnp.dot(oh, inj_ref[0], preferred_element_type=jnp.float32,
                precision=lax.Precision.HIGHEST)
    out_ref[0] = x_ref[0] * m + a


def _tc_kernel(inj3, msk3, x3, onehot):
    return pl.pallas_call(
        _tc_body,
        grid=(QT,),
        in_specs=[
            pl.BlockSpec((B, S), lambda q: (0, 0)),
            pl.BlockSpec((1, B, C), lambda q: (q, 0, 0)),
            pl.BlockSpec((1, S, C), lambda q: (q, 0, 0)),
            pl.BlockSpec((1, S, C), lambda q: (q, 0, 0)),
        ],
        out_specs=pl.BlockSpec((1, B, C), lambda q: (q, 0, 0)),
        out_shape=jax.ShapeDtypeStruct((Q, B, C), jnp.float32),
        compiler_params=pltpu.CompilerParams(
            dimension_semantics=("arbitrary",)),
    )(onehot, x3, inj3, msk3)


def kernel(inputs, injection_sites, masks, indices):
    inj3 = jnp.transpose(injection_sites, (1, 2, 0, 3)).reshape(Q, S, C)
    msk3 = jnp.transpose(masks, (1, 2, 0, 3)).reshape(Q, S, C)
    x3 = jnp.transpose(inputs, (1, 2, 0, 3)).reshape(Q, B, C)

    sc_out = _sc_kernel(inj3.reshape(Q * S, C), msk3.reshape(Q * S, C),
                        x3.reshape(NR, C), indices)

    onehot = (indices[:, None] == jnp.arange(S, dtype=jnp.int32)[None, :])
    onehot = onehot.astype(jnp.float32)
    tc_out = _tc_kernel(inj3, msk3, x3, onehot)

    out3 = lax.dynamic_update_slice(tc_out, sc_out.reshape(Q - QT, B, C),
                                    (QT, 0, 0))
    return jnp.transpose(out3.reshape(H, W, B, C), (2, 0, 1, 3))


# speedup vs baseline: 1.1870x; 1.1870x over previous
import jax
import jax.numpy as jnp
from jax import lax
from jax.experimental import pallas as pl
from jax.experimental.pallas import tpu as pltpu
from jax.experimental.pallas import tpu_sc as plsc

B = 1024
S = 256
H = 14
W = 14
C = 128
Q = H * W
QT = 136
NR = Q * B
NC = 2
NS = 16
NW = NC * NS
SCR = (Q - QT) * B
RPW = SCR // NW
RCH = 64
NSTEP = RPW // RCH
LANES = 16
CV = C // LANES


def _sc_kernel(inj2, msk2, x2, indices):
    mesh = plsc.VectorSubcoreMesh(core_axis_name="c", subcore_axis_name="s")

    def run(inj_hbm, msk_hbm, x_hbm, idx_hbm, out_hbm,
            idx_v, exp_v, x_v, m_v, a_v, o_v, sems, sem_wb):
        wid = lax.axis_index("s") * NC + lax.axis_index("c")
        base = QT * B + wid * RPW
        lbase = wid * RPW
        pltpu.sync_copy(idx_hbm, idx_v)

        lane = lax.iota(jnp.int32, LANES)

        @pl.loop(0, RPW, step=LANES)
        def _(t):
            r16 = base + t + lane
            q = lax.shift_right_logical(r16, 10)
            b = lax.bitwise_and(r16, 1023)
            v = plsc.load_gather(idx_v, [b])
            exp_v[pl.ds(t, LANES)] = lax.shift_left(q, 8) + v

        def issue(n, j):
            idxs = exp_v.at[pl.ds(n * RCH, RCH)]
            pltpu.async_copy(inj_hbm.at[idxs], a_v.at[j], sems.at[j, 0])
            pltpu.async_copy(msk_hbm.at[idxs], m_v.at[j], sems.at[j, 1])
            pltpu.async_copy(x_hbm.at[pl.ds(base + n * RCH, RCH)], x_v.at[j],
                             sems.at[j, 2])

        def wait_in(j):
            pltpu.make_async_copy(inj_hbm.at[pl.ds(0, RCH)], a_v.at[j],
                                  sems.at[j, 0]).wait()
            pltpu.make_async_copy(msk_hbm.at[pl.ds(0, RCH)], m_v.at[j],
                                  sems.at[j, 1]).wait()
            pltpu.make_async_copy(x_hbm.at[pl.ds(0, RCH)], x_v.at[j],
                                  sems.at[j, 2]).wait()

        def compute(j):
            xp, mp, ap = x_v.at[j], m_v.at[j], a_v.at[j]

            @pl.loop(0, RCH)
            def _(i):
                for c in range(CV):
                    sl = (i, pl.ds(c * LANES, LANES))
                    o_v.at[*sl][...] = (xp.at[*sl][...] * mp.at[*sl][...]
                                        + ap.at[*sl][...])

        def wb_start(n):
            pltpu.async_copy(o_v, out_hbm.at[pl.ds(lbase + n * RCH, RCH)],
                             sem_wb)

        def wb_wait():
            pltpu.make_async_copy(o_v, out_hbm.at[pl.ds(0, RCH)],
                                  sem_wb).wait()

        def step(n, j, wait_prev_wb, issue_next):
            wait_in(j)
            if wait_prev_wb:
                wb_wait()
            compute(j)
            wb_start(n)
            if issue_next:
                issue(n + 2, j)

        issue(0, 0)
        issue(1, 1)
        step(0, 0, False, True)
        step(1, 1, True, True)

        @pl.loop(2, NSTEP - 2, step=2)
        def _(n0):
            step(n0, 0, True, True)
            step(n0 + 1, 1, True, True)

        step(NSTEP - 2, 0, True, False)
        step(NSTEP - 1, 1, True, False)
        wb_wait()

    grid_kernel = pl.kernel(
        run,
        out_type=jax.ShapeDtypeStruct((SCR, C), jnp.float32),
        mesh=mesh,
        scratch_types=[
            pltpu.VMEM((B,), jnp.int32),
            pltpu.VMEM((RPW,), jnp.int32),
            pltpu.VMEM((2, RCH, C), jnp.float32),
            pltpu.VMEM((2, RCH, C), jnp.float32),
            pltpu.VMEM((2, RCH, C), jnp.float32),
            pltpu.VMEM((RCH, C), jnp.float32),
            pltpu.SemaphoreType.DMA((2, 3)),
            pltpu.SemaphoreType.DMA,
        ],
        compiler_params=pltpu.CompilerParams(use_tc_tiling_on_sc=True,
                                             needs_layout_passes=False),
    )
    return grid_kernel(inj2, msk2, x2, indices)


def _tc_body(oh_ref, x_ref, inj_hi, inj_lo, msk_hi, msk_lo, out_ref):
    oh = oh_ref[...]

    def gather(hi, lo):
        return (jnp.dot(oh, hi[0], preferred_element_type=jnp.float32)
                + jnp.dot(oh, lo[0], preferred_element_type=jnp.float32))

    out_ref[0] = x_ref[0] * gather(msk_hi, msk_lo) + gather(inj_hi, inj_lo)


def _tc_kernel(inj3, msk3, x3, onehot):
    def split(t):
        hi = t.astype(jnp.bfloat16)
        lo = (t - hi.astype(jnp.float32)).astype(jnp.bfloat16)
        return hi, lo

    inj_hi, inj_lo = split(inj3)
    msk_hi, msk_lo = split(msk3)
    tbl_spec = pl.BlockSpec((1, S, C), lambda q: (q, 0, 0))
    return pl.pallas_call(
        _tc_body,
        grid=(QT,),
        in_specs=[
            pl.BlockSpec((B, S), lambda q: (0, 0)),
            pl.BlockSpec((1, B, C), lambda q: (q, 0, 0)),
            tbl_spec, tbl_spec, tbl_spec, tbl_spec,
        ],
        out_specs=pl.BlockSpec((1, B, C), lambda q: (q, 0, 0)),
        out_shape=jax.ShapeDtypeStruct((Q, B, C), jnp.float32),
        compiler_params=pltpu.CompilerParams(
            dimension_semantics=("arbitrary",)),
    )(onehot, x3, inj_hi, inj_lo, msk_hi, msk_lo)


def kernel(inputs, injection_sites, masks, indices):
    inj3 = jnp.transpose(injection_sites, (1, 2, 0, 3)).reshape(Q, S, C)
    msk3 = jnp.transpose(masks, (1, 2, 0, 3)).reshape(Q, S, C)
    x3 = jnp.transpose(inputs, (1, 2, 0, 3)).reshape(Q, B, C)

    sc_out = _sc_kernel(inj3.reshape(Q * S, C), msk3.reshape(Q * S, C),
                        x3.reshape(NR, C), indices)

    onehot = (indices[:, None] == jnp.arange(S, dtype=jnp.int32)[None, :])
    onehot = onehot.astype(jnp.bfloat16)
    tc_out = _tc_kernel(inj3, msk3, x3, onehot)

    out3 = lax.dynamic_update_slice(tc_out, sc_out.reshape(Q - QT, B, C),
                                    (QT, 0, 0))
    return jnp.transpose(out3.reshape(H, W, B, C), (2, 0, 1, 3))


# speedup vs baseline: 1.6287x; 1.3722x over previous
import jax
import jax.numpy as jnp
from jax import lax
from jax.experimental import pallas as pl
from jax.experimental.pallas import tpu as pltpu
from jax.experimental.pallas import tpu_sc as plsc

B = 1024
S = 256
H = 14
W = 14
C = 128
Q = H * W
QT = 136
NR = Q * B
NC = 2
NS = 16
NW = NC * NS
SCR = (Q - QT) * B
RPW = SCR // NW
RCH = 64
NSTEP = RPW // RCH
LANES = 16
CV = C // LANES


def _sc_kernel(inj2, msk2, x2, indices):
    mesh = plsc.VectorSubcoreMesh(core_axis_name="c", subcore_axis_name="s")

    def run(inj_hbm, msk_hbm, x_hbm, idx_hbm, out_hbm,
            idx_v, exp_v, x_v, m_v, a_v, o_v, sems, sem_wb):
        wid = lax.axis_index("s") * NC + lax.axis_index("c")
        base = QT * B + wid * RPW
        lbase = wid * RPW
        pltpu.sync_copy(idx_hbm, idx_v)

        lane = lax.iota(jnp.int32, LANES)

        @pl.loop(0, RPW, step=LANES)
        def _(t):
            r16 = base + t + lane
            q = lax.shift_right_logical(r16, 10)
            b = lax.bitwise_and(r16, 1023)
            v = plsc.load_gather(idx_v, [b])
            exp_v[pl.ds(t, LANES)] = lax.shift_left(q, 8) + v

        def issue(n, j):
            idxs = exp_v.at[pl.ds(n * RCH, RCH)]
            pltpu.async_copy(inj_hbm.at[idxs], a_v.at[j], sems.at[j, 0])
            pltpu.async_copy(msk_hbm.at[idxs], m_v.at[j], sems.at[j, 1])
            pltpu.async_copy(x_hbm.at[pl.ds(base + n * RCH, RCH)], x_v.at[j],
                             sems.at[j, 2])

        def wait_in(j):
            pltpu.make_async_copy(inj_hbm.at[pl.ds(0, RCH)], a_v.at[j],
                                  sems.at[j, 0]).wait()
            pltpu.make_async_copy(msk_hbm.at[pl.ds(0, RCH)], m_v.at[j],
                                  sems.at[j, 1]).wait()
            pltpu.make_async_copy(x_hbm.at[pl.ds(0, RCH)], x_v.at[j],
                                  sems.at[j, 2]).wait()

        def compute(j):
            xp, mp, ap = x_v.at[j], m_v.at[j], a_v.at[j]

            @pl.loop(0, RCH)
            def _(i):
                for c in range(CV):
                    sl = (i, pl.ds(c * LANES, LANES))
                    o_v.at[*sl][...] = (xp.at[*sl][...] * mp.at[*sl][...]
                                        + ap.at[*sl][...])

        def wb_start(n):
            pltpu.async_copy(o_v, out_hbm.at[pl.ds(lbase + n * RCH, RCH)],
                             sem_wb)

        def wb_wait():
            pltpu.make_async_copy(o_v, out_hbm.at[pl.ds(0, RCH)],
                                  sem_wb).wait()

        def step(n, j, wait_prev_wb, issue_next):
            wait_in(j)
            if wait_prev_wb:
                wb_wait()
            compute(j)
            wb_start(n)
            if issue_next:
                issue(n + 2, j)

        issue(0, 0)
        issue(1, 1)
        step(0, 0, False, True)
        step(1, 1, True, True)

        @pl.loop(2, NSTEP - 2, step=2)
        def _(n0):
            step(n0, 0, True, True)
            step(n0 + 1, 1, True, True)

        step(NSTEP - 2, 0, True, False)
        step(NSTEP - 1, 1, True, False)
        wb_wait()

    grid_kernel = pl.kernel(
        run,
        out_type=jax.ShapeDtypeStruct((SCR, C), jnp.float32),
        mesh=mesh,
        scratch_types=[
            pltpu.VMEM((B,), jnp.int32),
            pltpu.VMEM((RPW,), jnp.int32),
            pltpu.VMEM((2, RCH, C), jnp.float32),
            pltpu.VMEM((2, RCH, C), jnp.float32),
            pltpu.VMEM((2, RCH, C), jnp.float32),
            pltpu.VMEM((RCH, C), jnp.float32),
            pltpu.SemaphoreType.DMA((2, 3)),
            pltpu.SemaphoreType.DMA,
        ],
        compiler_params=pltpu.CompilerParams(use_tc_tiling_on_sc=True,
                                             needs_layout_passes=False),
    )
    return grid_kernel(inj2, msk2, x2, indices)


def _tc_body(idx_ref, x_ref, inj_ref, msk_ref, out_ref, oh_ref):
    @pl.when(pl.program_id(0) == 0)
    def _():
        iota = lax.broadcasted_iota(jnp.int32, (B, S), 1)
        oh_ref[...] = (idx_ref[...] == iota).astype(jnp.bfloat16)

    cat = jnp.concatenate(
        [msk_ref[0].astype(jnp.bfloat16), inj_ref[0].astype(jnp.bfloat16)],
        axis=1)
    r = jnp.dot(oh_ref[...], cat, preferred_element_type=jnp.float32)
    out_ref[0] = x_ref[0] * r[:, :C] + r[:, C:]


def _tc_kernel(inj3, msk3, x3, idx2):
    tbl_spec = pl.BlockSpec((1, S, C), lambda q: (q, 0, 0))
    return pl.pallas_call(
        _tc_body,
        grid=(QT,),
        in_specs=[
            pl.BlockSpec((B, 1), lambda q: (0, 0)),
            pl.BlockSpec((1, B, C), lambda q: (q, 0, 0)),
            tbl_spec, tbl_spec,
        ],
        out_specs=pl.BlockSpec((1, B, C), lambda q: (q, 0, 0)),
        out_shape=jax.ShapeDtypeStruct((Q, B, C), jnp.float32),
        scratch_shapes=[pltpu.VMEM((B, S), jnp.bfloat16)],
        compiler_params=pltpu.CompilerParams(
            dimension_semantics=("arbitrary",)),
    )(idx2, x3, inj3, msk3)


def kernel(inputs, injection_sites, masks, indices):
    inj3 = jnp.transpose(injection_sites, (1, 2, 0, 3)).reshape(Q, S, C)
    msk3 = jnp.transpose(masks, (1, 2, 0, 3)).reshape(Q, S, C)
    x3 = jnp.transpose(inputs, (1, 2, 0, 3)).reshape(Q, B, C)

    sc_out = _sc_kernel(inj3.reshape(Q * S, C), msk3.reshape(Q * S, C),
                        x3.reshape(NR, C), indices)

    tc_out = _tc_kernel(inj3, msk3, x3, indices[:, None])

    out3 = lax.dynamic_update_slice(tc_out, sc_out.reshape(Q - QT, B, C),
                                    (QT, 0, 0))
    return jnp.transpose(out3.reshape(H, W, B, C), (2, 0, 1, 3))


# speedup vs baseline: 1.7011x; 1.0444x over previous
import jax
import jax.numpy as jnp
from jax import lax
from jax.experimental import pallas as pl
from jax.experimental.pallas import tpu as pltpu
from jax.experimental.pallas import tpu_sc as plsc

B = 1024
S = 256
H = 14
W = 14
C = 128
Q = H * W
QT = 104
NR = Q * B
NC = 2
NS = 16
NW = NC * NS
SCR = (Q - QT) * B
RPW = SCR // NW
RCH = 64
NSTEP = RPW // RCH
LANES = 16
CV = C // LANES


def _sc_kernel(inj2, msk2, x2, indices):
    mesh = plsc.VectorSubcoreMesh(core_axis_name="c", subcore_axis_name="s")

    def run(inj_hbm, msk_hbm, x_hbm, idx_hbm, out_hbm,
            idx_v, exp_v, x_v, m_v, a_v, o_v, sems, sem_wb):
        wid = lax.axis_index("s") * NC + lax.axis_index("c")
        base = QT * B + wid * RPW
        lbase = wid * RPW
        pltpu.sync_copy(idx_hbm, idx_v)

        lane = lax.iota(jnp.int32, LANES)

        @pl.loop(0, RPW, step=LANES)
        def _(t):
            r16 = base + t + lane
            q = lax.shift_right_logical(r16, 10)
            b = lax.bitwise_and(r16, 1023)
            v = plsc.load_gather(idx_v, [b])
            exp_v[pl.ds(t, LANES)] = lax.shift_left(q, 8) + v

        def issue(n, j):
            idxs = exp_v.at[pl.ds(n * RCH, RCH)]
            pltpu.async_copy(inj_hbm.at[idxs], a_v.at[j], sems.at[j, 0])
            pltpu.async_copy(msk_hbm.at[idxs], m_v.at[j], sems.at[j, 1])
            pltpu.async_copy(x_hbm.at[pl.ds(base + n * RCH, RCH)], x_v.at[j],
                             sems.at[j, 2])

        def wait_in(j):
            pltpu.make_async_copy(inj_hbm.at[pl.ds(0, RCH)], a_v.at[j],
                                  sems.at[j, 0]).wait()
            pltpu.make_async_copy(msk_hbm.at[pl.ds(0, RCH)], m_v.at[j],
                                  sems.at[j, 1]).wait()
            pltpu.make_async_copy(x_hbm.at[pl.ds(0, RCH)], x_v.at[j],
                                  sems.at[j, 2]).wait()

        def compute(j):
            xp, mp, ap = x_v.at[j], m_v.at[j], a_v.at[j]

            @pl.loop(0, RCH)
            def _(i):
                for c in range(CV):
                    sl = (i, pl.ds(c * LANES, LANES))
                    o_v.at[*sl][...] = (xp.at[*sl][...] * mp.at[*sl][...]
                                        + ap.at[*sl][...])

        def wb_start(n):
            pltpu.async_copy(o_v, out_hbm.at[pl.ds(lbase + n * RCH, RCH)],
                             sem_wb)

        def wb_wait():
            pltpu.make_async_copy(o_v, out_hbm.at[pl.ds(0, RCH)],
                                  sem_wb).wait()

        def step(n, j, wait_prev_wb, issue_next):
            wait_in(j)
            if wait_prev_wb:
                wb_wait()
            compute(j)
            wb_start(n)
            if issue_next:
                issue(n + 2, j)

        issue(0, 0)
        issue(1, 1)
        step(0, 0, False, True)
        step(1, 1, True, True)

        @pl.loop(2, NSTEP - 2, step=2)
        def _(n0):
            step(n0, 0, True, True)
            step(n0 + 1, 1, True, True)

        step(NSTEP - 2, 0, True, False)
        step(NSTEP - 1, 1, True, False)
        wb_wait()

    grid_kernel = pl.kernel(
        run,
        out_type=jax.ShapeDtypeStruct((SCR, C), jnp.float32),
        mesh=mesh,
        scratch_types=[
            pltpu.VMEM((B,), jnp.int32),
            pltpu.VMEM((RPW,), jnp.int32),
            pltpu.VMEM((2, RCH, C), jnp.float32),
            pltpu.VMEM((2, RCH, C), jnp.float32),
            pltpu.VMEM((2, RCH, C), jnp.float32),
            pltpu.VMEM((RCH, C), jnp.float32),
            pltpu.SemaphoreType.DMA((2, 3)),
            pltpu.SemaphoreType.DMA,
        ],
        compiler_params=pltpu.CompilerParams(use_tc_tiling_on_sc=True,
                                             needs_layout_passes=False),
    )
    return grid_kernel(inj2, msk2, x2, indices)


def _tc_body(idx_ref, x_ref, inj_ref, msk_ref, out_ref, oh_ref):
    @pl.when(pl.program_id(0) == 0)
    def _():
        iota = lax.broadcasted_iota(jnp.int32, (B, S), 1)
        oh_ref[...] = (idx_ref[...] == iota).astype(jnp.bfloat16)

    cat = jnp.concatenate(
        [msk_ref[0].astype(jnp.bfloat16), inj_ref[0].astype(jnp.bfloat16)],
        axis=1)
    r = jnp.dot(oh_ref[...], cat, preferred_element_type=jnp.float32)
    out_ref[0] = x_ref[0] * r[:, :C] + r[:, C:]


def _tc_kernel(inj3, msk3, x3, idx2):
    tbl_spec = pl.BlockSpec((1, S, C), lambda q: (q, 0, 0))
    return pl.pallas_call(
        _tc_body,
        grid=(QT,),
        in_specs=[
            pl.BlockSpec((B, 1), lambda q: (0, 0)),
            pl.BlockSpec((1, B, C), lambda q: (q, 0, 0)),
            tbl_spec, tbl_spec,
        ],
        out_specs=pl.BlockSpec((1, B, C), lambda q: (q, 0, 0)),
        out_shape=jax.ShapeDtypeStruct((Q, B, C), jnp.float32),
        scratch_shapes=[pltpu.VMEM((B, S), jnp.bfloat16)],
        compiler_params=pltpu.CompilerParams(
            dimension_semantics=("arbitrary",)),
    )(idx2, x3, inj3, msk3)


def kernel(inputs, injection_sites, masks, indices):
    inj3 = jnp.transpose(injection_sites, (1, 2, 0, 3)).reshape(Q, S, C)
    msk3 = jnp.transpose(masks, (1, 2, 0, 3)).reshape(Q, S, C)
    x3 = jnp.transpose(inputs, (1, 2, 0, 3)).reshape(Q, B, C)

    sc_out = _sc_kernel(inj3.reshape(Q * S, C), msk3.reshape(Q * S, C),
                        x3.reshape(NR, C), indices)

    tc_out = _tc_kernel(inj3, msk3, x3, indices[:, None])

    out3 = lax.dynamic_update_slice(tc_out, sc_out.reshape(Q - QT, B, C),
                                    (QT, 0, 0))
    return jnp.transpose(out3.reshape(H, W, B, C), (2, 0, 1, 3))


# speedup vs baseline: 1.7593x; 1.0342x over previous
import jax
import jax.numpy as jnp
from jax import lax
from jax.experimental import pallas as pl
from jax.experimental.pallas import tpu as pltpu
from jax.experimental.pallas import tpu_sc as plsc

B = 1024
S = 256
H = 14
W = 14
C = 128
Q = H * W
NR = Q * B
NC = 2
NS = 16
NW = NC * NS
RPW = NR // NW
RCH = 112
NSTEP = RPW // RCH
LANES = 16
CV = C // LANES


def kernel(inputs, injection_sites, masks, indices):
    inj2 = jnp.transpose(injection_sites, (1, 2, 0, 3)).reshape(Q * S, C)
    msk2 = jnp.transpose(masks, (1, 2, 0, 3)).reshape(Q * S, C)
    x2 = jnp.transpose(inputs, (1, 2, 0, 3)).reshape(NR, C)

    mesh = plsc.VectorSubcoreMesh(core_axis_name="c", subcore_axis_name="s")

    def run(inj_hbm, msk_hbm, x_hbm, idx_hbm, out_hbm,
            idx_v, exp_v, x_v, m_v, a_v, o_v, sems, sem_wb):
        wid = lax.axis_index("s") * NC + lax.axis_index("c")
        base = wid * RPW
        pltpu.sync_copy(idx_hbm, idx_v)

        lane = lax.iota(jnp.int32, LANES)

        @pl.loop(0, RPW, step=LANES)
        def _(t):
            r16 = base + t + lane
            q = lax.shift_right_logical(r16, 10)
            b = lax.bitwise_and(r16, 1023)
            v = plsc.load_gather(idx_v, [b])
            exp_v[pl.ds(t, LANES)] = lax.shift_left(q, 8) + v

        def issue(n, j):
            idxs = exp_v.at[pl.ds(n * RCH, RCH)]
            r0 = base + n * RCH
            pltpu.async_copy(inj_hbm.at[idxs], a_v.at[j], sems.at[j, 0])
            pltpu.async_copy(msk_hbm.at[idxs], m_v.at[j], sems.at[j, 1])
            pltpu.async_copy(x_hbm.at[pl.ds(r0, RCH)], x_v.at[j], sems.at[j, 2])

        def wait_in(j):
            pltpu.make_async_copy(inj_hbm.at[pl.ds(0, RCH)], a_v.at[j],
                                  sems.at[j, 0]).wait()
            pltpu.make_async_copy(msk_hbm.at[pl.ds(0, RCH)], m_v.at[j],
                                  sems.at[j, 1]).wait()
            pltpu.make_async_copy(x_hbm.at[pl.ds(0, RCH)], x_v.at[j],
                                  sems.at[j, 2]).wait()

        def compute(j):
            xp, mp, ap = x_v.at[j], m_v.at[j], a_v.at[j]

            @pl.loop(0, RCH)
            def _(i):
                for c in range(CV):
                    sl = (i, pl.ds(c * LANES, LANES))
                    o_v.at[*sl][...] = (xp.at[*sl][...] * mp.at[*sl][...]
                                        + ap.at[*sl][...])

        def wb_start(n):
            r0 = base + n * RCH
            pltpu.async_copy(o_v, out_hbm.at[pl.ds(r0, RCH)], sem_wb)

        def wb_wait():
            pltpu.make_async_copy(o_v, out_hbm.at[pl.ds(0, RCH)],
                                  sem_wb).wait()

        def step(n, j, wait_prev_wb, issue_next):
            wait_in(j)
            if wait_prev_wb:
                wb_wait()
            compute(j)
            wb_start(n)
            if issue_next:
                issue(n + 2, j)

        issue(0, 0)
        issue(1, 1)
        step(0, 0, False, True)
        step(1, 1, True, True)

        @pl.loop(2, NSTEP - 2, step=2)
        def _(n0):
            step(n0, 0, True, True)
            step(n0 + 1, 1, True, True)

        step(NSTEP - 2, 0, True, False)
        step(NSTEP - 1, 1, True, False)
        wb_wait()

    grid_kernel = pl.kernel(
        run,
        out_type=jax.ShapeDtypeStruct((NR, C), jnp.float32),
        mesh=mesh,
        scratch_types=[
            pltpu.VMEM((B,), jnp.int32),
            pltpu.VMEM((RPW,), jnp.int32),
            pltpu.VMEM((2, RCH, C), jnp.float32),
            pltpu.VMEM((2, RCH, C), jnp.float32),
            pltpu.VMEM((2, RCH, C), jnp.float32),
            pltpu.VMEM((RCH, C), jnp.float32),
            pltpu.SemaphoreType.DMA((2, 3)),
            pltpu.SemaphoreType.DMA,
        ],
        compiler_params=pltpu.CompilerParams(use_tc_tiling_on_sc=True,
                                             needs_layout_passes=False),
    )
    out2 = grid_kernel(inj2, msk2, x2, indices)
    return jnp.transpose(out2.reshape(H, W, B, C), (2, 0, 1, 3))
